# baseline (device time: 40204 ns/iter reference)
import jax
import jax.numpy as jnp
from jax import lax
from jax.experimental import pallas as pl
from jax.experimental.pallas import tpu as pltpu

N_DEV = 16
CAPACITY = 12.0


def kernel(x, router_W, route_idx, expert_W):
    m, d = x.shape
    e_local, _, h = expert_W.shape
    chunk = m // N_DEV

    def body(x_ref, rW_ref, idx_ref, w_ref, out_ref,
             partial_ref, rs_ref, sendA, recvA, sendB, recvB):
        my = lax.axis_index("i")

        xb = x_ref[:, :].astype(jnp.bfloat16)
        ridx = idx_ref[:, :]
        eids = my * e_local + lax.broadcasted_iota(jnp.int32, (m, e_local), 1)
        onehot = ridx == eids
        row = lax.broadcasted_iota(jnp.int32, (m, m), 0)
        col = lax.broadcasted_iota(jnp.int32, (m, m), 1)
        tri = (row >= col).astype(jnp.bfloat16)
        cnt = jnp.dot(tri, onehot.astype(jnp.bfloat16),
                      preferred_element_type=jnp.float32)
        gate = jnp.logical_and(onehot, cnt <= CAPACITY).astype(jnp.bfloat16)

        acc = jnp.zeros((m, h), jnp.float32)
        for le in range(e_local):
            xm = xb * gate[:, le:le + 1]
            w = w_ref[le, :, :].astype(jnp.bfloat16)
            acc = acc + jnp.dot(xm, w, preferred_element_type=jnp.float32)
        partial_ref[:, :] = acc.astype(jnp.bfloat16)

        rs_ref[0, :, :] = partial_ref[pl.ds(my * chunk, chunk), :]

        sends_a = []
        for k in range(1, N_DEV):
            j = (my + k) % N_DEV
            slot = N_DEV - k
            r = pltpu.make_async_remote_copy(
                src_ref=partial_ref.at[pl.ds(j * chunk, chunk), :],
                dst_ref=rs_ref.at[slot],
                send_sem=sendA.at[k],
                recv_sem=recvA.at[slot],
                device_id=(j,),
                device_id_type=pl.DeviceIdType.MESH,
            )
            r.start()
            sends_a.append(r)

        for s in range(1, N_DEV):
            pltpu.make_async_remote_copy(
                src_ref=rs_ref.at[0],
                dst_ref=rs_ref.at[s],
                send_sem=sendA.at[0],
                recv_sem=recvA.at[s],
                device_id=(my,),
                device_id_type=pl.DeviceIdType.MESH,
            ).wait_recv()

        reduced = jnp.sum(rs_ref[:, :, :].astype(jnp.float32), axis=0)
        out_ref[pl.ds(my * chunk, chunk), :] = reduced.astype(jnp.bfloat16)

        sends_b = []
        for k in range(1, N_DEV):
            j = (my + k) % N_DEV
            slot = N_DEV - k
            r = pltpu.make_async_remote_copy(
                src_ref=out_ref.at[pl.ds(my * chunk, chunk), :],
                dst_ref=out_ref.at[pl.ds(my * chunk, chunk), :],
                send_sem=sendB.at[k],
                recv_sem=recvB.at[slot],
                device_id=(j,),
                device_id_type=pl.DeviceIdType.MESH,
            )
            r.start()
            sends_b.append(r)

        for s in range(1, N_DEV):
            origin = (my + s) % N_DEV
            pltpu.make_async_remote_copy(
                src_ref=rs_ref.at[0],
                dst_ref=out_ref.at[pl.ds(origin * chunk, chunk), :],
                send_sem=sendB.at[0],
                recv_sem=recvB.at[s],
                device_id=(my,),
                device_id_type=pl.DeviceIdType.MESH,
            ).wait_recv()

        for r in sends_a + sends_b:
            r.wait_send()

    return pl.pallas_call(
        body,
        out_shape=jax.ShapeDtypeStruct((m, h), jnp.bfloat16),
        in_specs=[pl.BlockSpec(memory_space=pltpu.VMEM)] * 4,
        out_specs=pl.BlockSpec(memory_space=pltpu.VMEM),
        scratch_shapes=[
            pltpu.VMEM((m, h), jnp.bfloat16),
            pltpu.VMEM((N_DEV, chunk, h), jnp.bfloat16),
            pltpu.SemaphoreType.DMA((N_DEV,)),
            pltpu.SemaphoreType.DMA((N_DEV,)),
            pltpu.SemaphoreType.DMA((N_DEV,)),
            pltpu.SemaphoreType.DMA((N_DEV,)),
        ],
    )(x, router_W, route_idx, expert_W)


# device time: 26622 ns/iter; 1.5102x vs baseline; 1.5102x over previous
import jax
import jax.numpy as jnp
from jax import lax
from jax.experimental import pallas as pl
from jax.experimental.pallas import tpu as pltpu

N_DEV = 16
CAP = 12


def kernel(x, router_W, route_idx, expert_W):
    m, d = x.shape
    e_local, _, h = expert_W.shape
    n_slots = N_DEV * e_local * CAP
    blk = e_local * CAP

    def body(x_ref, rW_ref, idx_ref, w_ref, out_ref,
             blocks_ref, send_sems, recv_sems):
        my = lax.axis_index("i")

        ridx = idx_ref[:, :]
        e_all = lax.broadcasted_iota(jnp.int32, (m, N_DEV * e_local), 1)
        onehot = (ridx == e_all).astype(jnp.bfloat16)
        row = lax.broadcasted_iota(jnp.int32, (m, m), 0)
        col = lax.broadcasted_iota(jnp.int32, (m, m), 1)
        tri = (row >= col).astype(jnp.bfloat16)
        cnt = jnp.dot(tri, onehot, preferred_element_type=jnp.float32)
        cnt_tok = jnp.sum(cnt * onehot.astype(jnp.float32), axis=1,
                          keepdims=True).astype(jnp.int32)
        slot = ridx * CAP + cnt_tok - 1
        slot = jnp.where(cnt_tok <= CAP, slot, -1)

        slot_local = lax.broadcasted_iota(jnp.int32, (m, blk), 1) + my * blk
        gt = (slot == slot_local).astype(jnp.bfloat16)
        xb = x_ref[:, :].astype(jnp.bfloat16)
        xc = lax.dot_general(gt, xb, (((0,), (0,)), ((), ())),
                             preferred_element_type=jnp.float32)
        xc = xc.astype(jnp.bfloat16)
        parts = []
        for le in range(e_local):
            w = w_ref[le, :, :].astype(jnp.bfloat16)
            parts.append(jnp.dot(xc[le * CAP:(le + 1) * CAP, :], w,
                                 preferred_element_type=jnp.float32))
        block = jnp.concatenate(parts, axis=0).astype(jnp.bfloat16)
        blocks_ref[pl.ds(my * blk, blk), :] = block

        sends = []
        for k in range(1, N_DEV):
            j = (my + k) % N_DEV
            r = pltpu.make_async_remote_copy(
                src_ref=blocks_ref.at[pl.ds(my * blk, blk), :],
                dst_ref=blocks_ref.at[pl.ds(my * blk, blk), :],
                send_sem=send_sems.at[k],
                recv_sem=recv_sems.at[N_DEV - k],
                device_id=(j,),
                device_id_type=pl.DeviceIdType.MESH,
            )
            r.start()
            sends.append(r)

        slot_all = lax.broadcasted_iota(jnp.int32, (m, n_slots), 1)
        p_mat = (slot == slot_all).astype(jnp.bfloat16)

        for s in range(1, N_DEV):
            origin = (my + s) % N_DEV
            pltpu.make_async_remote_copy(
                src_ref=blocks_ref.at[pl.ds(0, blk), :],
                dst_ref=blocks_ref.at[pl.ds(origin * blk, blk), :],
                send_sem=send_sems.at[0],
                recv_sem=recv_sems.at[s],
                device_id=(my,),
                device_id_type=pl.DeviceIdType.MESH,
            ).wait_recv()

        out_ref[:, :] = jnp.dot(
            p_mat, blocks_ref[:, :], preferred_element_type=jnp.float32
        ).astype(jnp.bfloat16)

        for r in sends:
            r.wait_send()

    return pl.pallas_call(
        body,
        out_shape=jax.ShapeDtypeStruct((m, h), jnp.bfloat16),
        in_specs=[pl.BlockSpec(memory_space=pltpu.VMEM)] * 4,
        out_specs=pl.BlockSpec(memory_space=pltpu.VMEM),
        scratch_shapes=[
            pltpu.VMEM((n_slots, h), jnp.bfloat16),
            pltpu.SemaphoreType.DMA((N_DEV,)),
            pltpu.SemaphoreType.DMA((N_DEV,)),
        ],
    )(x, router_W, route_idx, expert_W)
